# baseline (speedup 1.0000x reference)
import functools

import jax
import jax.numpy as jnp
from jax import lax
from jax.experimental import pallas as pl
from jax.experimental.pallas import tpu as pltpu
from jax.experimental.pallas import tpu_sc as plsc

E = 8
TOPK = 2
DM = 1024
DF = 4096
CAP = 640
NSLOT = E * CAP
NC = 2
NS = 16
NW = NC * NS
SPW = NSLOT // NW
GCH = SPW // 2
L = 16

FT = 512
CT = 128
KT = 2048


NROW = 5248
TRASH = NSLOT
TPW = 2048 // NW


def _gate_body(x_ref, wg_ref, d1_ref, d2_ref, t1_ref, t2_ref, k1_ref, k2_ref,
               g1_ref, g2_ref, sw_ref, xmean_ref, x16_ref):
    xf = x_ref[...]
    T = xf.shape[0]
    logits = jnp.dot(xf, wg_ref[...], preferred_element_type=jnp.float32)
    m = jnp.max(logits, axis=-1, keepdims=True)
    ex = jnp.exp(logits - m)
    sm = ex / jnp.sum(ex, axis=-1, keepdims=True)
    iotaE = lax.broadcasted_iota(jnp.int32, (T, E), 1)
    g1v = jnp.max(sm, axis=-1, keepdims=True)
    e1 = jnp.min(jnp.where(sm >= g1v, iotaE, E), axis=-1, keepdims=True)
    sm2 = jnp.where(iotaE == e1, -1.0, sm)
    g2v = jnp.max(sm2, axis=-1, keepdims=True)
    e2 = jnp.min(jnp.where(sm2 >= g2v, iotaE, E), axis=-1, keepdims=True)
    ssum = g1v + g2v + 1e-9
    g1n = g1v / ssum
    g2n = g2v / ssum
    mask1 = (iotaE == e1).astype(jnp.float32)
    mask2 = (iotaE == e2).astype(jnp.float32)
    both = jnp.concatenate([mask1, mask2], axis=1)
    r = lax.broadcasted_iota(jnp.int32, (T, T), 0)
    c = lax.broadcasted_iota(jnp.int32, (T, T), 1)
    tri = (r >= c).astype(jnp.float32)
    cum = jnp.dot(tri, both, preferred_element_type=jnp.float32)
    cum1 = cum[:, :E]
    cum2 = cum[:, E:]
    n1 = cum1[T - 1:T, :]
    loc1 = cum1 - 1.0
    loc2 = cum2 - 1.0 + n1
    m1k = mask1 * (loc1 < CAP).astype(jnp.float32)
    m2k = mask2 * (loc2 < CAP).astype(jnp.float32)
    pos1 = jnp.sum(loc1 * m1k, axis=-1, keepdims=True).astype(jnp.int32)
    pos2 = jnp.sum(loc2 * m2k, axis=-1, keepdims=True).astype(jnp.int32)
    kept1 = jnp.sum(m1k, axis=-1, keepdims=True)
    kept2 = jnp.sum(m2k, axis=-1, keepdims=True)
    g1 = g1n * kept1
    g2 = g2n * kept2
    d1_ref[...] = e1 * CAP + pos1
    d2_ref[...] = e2 * CAP + pos2
    t1_ref[...] = pos1 * E + e1
    t2_ref[...] = pos2 * E + e2
    k1_ref[...] = kept1.astype(jnp.int32)
    k2_ref[...] = kept2.astype(jnp.int32)
    g1_ref[...] = g1
    g2_ref[...] = g2
    sw_ref[...] = jnp.sum(m1k * g1 + m2k * g2, axis=0, keepdims=True)
    xmean_ref[...] = jnp.sum(xf, axis=0, keepdims=True) * (1.0 / T)
    x16_ref[...] = xf.astype(jnp.bfloat16)


def _gate(xf, Wg):
    T = xf.shape[0]
    f32, i32 = jnp.float32, jnp.int32
    outs = [((T, 1), i32), ((T, 1), i32), ((T, 1), i32), ((T, 1), i32),
            ((T, 1), i32), ((T, 1), i32), ((T, 1), f32), ((T, 1), f32),
            ((1, E), f32), ((1, DM), f32), ((T, DM), jnp.bfloat16)]
    return pl.pallas_call(
        _gate_body,
        out_shape=[jax.ShapeDtypeStruct(s, d) for s, d in outs],
    )(xf, Wg)


def _dispatch_body(d1_hbm, d2_hbm, t1_hbm, t2_hbm, k1_hbm, k2_hbm,
                   g1_hbm, g2_hbm, x16_hbm,
                   buf_hbm, w_hbm,
                   idx1v, idx2v, k1v, k2v, rowsv,
                   dt1v, dt2v, kk1v, kk2v, gg1v, gg2v, wv,
                   sem1, sem2):
    cid = lax.axis_index("c")
    sid = lax.axis_index("s")
    wid = sid * NC + cid
    t0 = wid * TPW

    pltpu.sync_copy(x16_hbm.at[pl.ds(t0, TPW)], rowsv)
    pltpu.sync_copy(d1_hbm.at[pl.ds(t0, TPW)], idx1v)
    pltpu.sync_copy(d2_hbm.at[pl.ds(t0, TPW)], idx2v)
    pltpu.sync_copy(k1_hbm.at[pl.ds(t0, TPW)], k1v)
    pltpu.sync_copy(k2_hbm.at[pl.ds(t0, TPW)], k2v)
    for j in range(TPW // L):
        sl = pl.ds(j * L, L)
        idx1v[sl] = jnp.where(k1v[sl] != 0, idx1v[sl], TRASH)
        idx2v[sl] = jnp.where(k2v[sl] != 0, idx2v[sl], TRASH)
    cp1 = pltpu.async_copy(rowsv, buf_hbm.at[idx1v], sem1)
    cp2 = pltpu.async_copy(rowsv, buf_hbm.at[idx2v], sem2)

    @pl.when(jnp.logical_and(cid == 0, sid == 0))
    def _():
        T = dt1v.shape[0]
        pltpu.sync_copy(t1_hbm, dt1v)
        pltpu.sync_copy(t2_hbm, dt2v)
        pltpu.sync_copy(k1_hbm, kk1v)
        pltpu.sync_copy(k2_hbm, kk2v)
        pltpu.sync_copy(g1_hbm, gg1v)
        pltpu.sync_copy(g2_hbm, gg2v)
        zf = jnp.zeros((L,), jnp.float32)

        def zero_body(i, _):
            wv[pl.ds(i * L, L)] = zf
            return 0

        lax.fori_loop(0, NSLOT // L, zero_body, 0)

        def scat_body(i, _):
            sl = pl.ds(i * L, L)
            plsc.store_scatter(wv, [dt1v[sl]], gg1v[sl], mask=kk1v[sl] != 0)
            plsc.store_scatter(wv, [dt2v[sl]], gg2v[sl], mask=kk2v[sl] != 0)
            return 0

        lax.fori_loop(0, T // L, scat_body, 0)
        pltpu.sync_copy(wv, w_hbm)

    cp1.wait()
    cp2.wait()


def _dispatch(d1, d2, t1, t2, k1, k2, g1, g2, x16):
    T = x16.shape[0]
    mesh = plsc.VectorSubcoreMesh(core_axis_name="c", subcore_axis_name="s")
    f32, i32 = jnp.float32, jnp.int32
    kern = pl.kernel(
        _dispatch_body,
        out_type=[jax.ShapeDtypeStruct((NROW, DM // 2), i32),
                  jax.ShapeDtypeStruct((NSLOT,), f32)],
        mesh=mesh,
        scratch_types=[
            pltpu.VMEM((TPW,), i32), pltpu.VMEM((TPW,), i32),
            pltpu.VMEM((TPW,), i32), pltpu.VMEM((TPW,), i32),
            pltpu.VMEM((TPW, DM // 2), i32),
            pltpu.VMEM((T,), i32), pltpu.VMEM((T,), i32),
            pltpu.VMEM((T,), i32), pltpu.VMEM((T,), i32),
            pltpu.VMEM((T,), f32), pltpu.VMEM((T,), f32),
            pltpu.VMEM((NSLOT,), f32),
            pltpu.SemaphoreType.DMA, pltpu.SemaphoreType.DMA,
        ],
        compiler_params=pltpu.CompilerParams(needs_layout_passes=False),
    )
    return kern(d1, d2, t1, t2, k1, k2, g1, g2, x16)


def _ffn1_body(buf_ref, w1_ref, b1_ref, wT_ref, hw_ref, w1bf_ref):
    e = pl.program_id(0)
    ft = pl.program_id(1)
    ct = pl.program_id(2)

    @pl.when(jnp.logical_and(ft == 0, ct == 0))
    def _():
        hw_ref[...] = jnp.zeros_like(hw_ref)

    @pl.when(ct == 0)
    def _():
        w1bf_ref[...] = w1_ref[0].astype(jnp.bfloat16)

    x = buf_ref[...]
    h = jnp.dot(x, w1bf_ref[...], preferred_element_type=jnp.float32)
    b1all = b1_ref[:, pl.ds(ft * FT, FT)]
    row = lax.broadcasted_iota(jnp.int32, (E, FT), 0)
    b1row = jnp.sum(jnp.where(row == e, b1all, 0.0), axis=0, keepdims=True)
    h = jnp.maximum(h + b1row, 0.0)
    wall = wT_ref[pl.ds(ct * CT, CT), :]
    lane = lax.broadcasted_iota(jnp.int32, (CT, E), 1)
    wv = jnp.sum(jnp.where(lane == e, wall, 0.0), axis=1, keepdims=True)
    red = jnp.sum(jnp.where(wv > 0.0, h * wv, 0.0), axis=0, keepdims=True)
    hw_ref[0, pl.ds(ft, 1), :] += red


def _ffn1(buf2d, W1, b1, wT):
    grid = (E, DF // FT, CAP // CT)
    return pl.pallas_call(
        _ffn1_body,
        grid=grid,
        in_specs=[
            pl.BlockSpec((CT, DM), lambda e, f, c: (e * (CAP // CT) + c, 0)),
            pl.BlockSpec((1, DM, FT), lambda e, f, c: (e, 0, f)),
            pl.BlockSpec((E, DF), lambda e, f, c: (0, 0)),
            pl.BlockSpec((CAP, E), lambda e, f, c: (0, 0)),
        ],
        out_specs=pl.BlockSpec((1, DF // FT, FT), lambda e, f, c: (e, 0, 0)),
        out_shape=jax.ShapeDtypeStruct((E, DF // FT, FT), jnp.float32),
        scratch_shapes=[pltpu.VMEM((DM, FT), jnp.bfloat16)],
    )(buf2d, W1, b1, wT)


def _ffn2_body(hw_ref, w2_ref, xmean_ref, sw_ref, b2_ref, y_ref, out_ref,
               acc_ref):
    k = pl.program_id(0)
    nk = pl.num_programs(0)

    @pl.when(k == 0)
    def _():
        acc_ref[...] = jnp.zeros_like(acc_ref)

    acc_ref[...] += jnp.dot(hw_ref[...], w2_ref[...],
                            preferred_element_type=jnp.float32)

    @pl.when(k == nk - 1)
    def _():
        T = 2048.0
        bias = jnp.dot(sw_ref[...], b2_ref[...],
                       preferred_element_type=jnp.float32)
        sent = xmean_ref[...] + (acc_ref[...] + bias) * (1.0 / T)
        mm = jnp.max(sent)
        lse = jnp.log(jnp.sum(jnp.exp(sent - mm))) + mm
        col = lax.broadcasted_iota(jnp.int32, (1, DM), 1)
        tgt = jnp.sum(jnp.where(col == y_ref[0, 0], sent, 0.0))
        out_ref[0, 0] = lse - tgt


def _ffn2_loss(hwf, W2f, xmean, sw, b2, y2):
    nk = hwf.shape[1] // KT
    return pl.pallas_call(
        _ffn2_body,
        grid=(nk,),
        in_specs=[
            pl.BlockSpec((1, KT), lambda k: (0, k)),
            pl.BlockSpec((KT, DM), lambda k: (k, 0)),
            pl.BlockSpec((1, DM), lambda k: (0, 0)),
            pl.BlockSpec((1, E), lambda k: (0, 0)),
            pl.BlockSpec((E, DM), lambda k: (0, 0)),
            pl.BlockSpec(memory_space=pltpu.SMEM),
        ],
        out_specs=pl.BlockSpec(memory_space=pltpu.SMEM),
        out_shape=jax.ShapeDtypeStruct((1, 1), jnp.float32),
        scratch_shapes=[pltpu.VMEM((1, DM), jnp.float32)],
    )(hwf, W2f, xmean, sw, b2, y2)


def kernel(x, y, Wg, W1, b1, W2, b2):
    B, S, _ = x.shape
    T = B * S
    xf = x.reshape(T, DM)
    d1, d2, t1, t2, k1, k2, g1, g2, sw, xmean, x16 = _gate(xf, Wg)
    x16i = lax.bitcast_convert_type(x16.reshape(T, DM // 2, 2), jnp.int32)
    buf, w = _dispatch(d1.reshape(T), d2.reshape(T), t1.reshape(T),
                       t2.reshape(T), k1.reshape(T), k2.reshape(T),
                       g1.reshape(T), g2.reshape(T), x16i)
    buf16 = lax.bitcast_convert_type(buf, jnp.bfloat16).reshape(NROW, DM)
    wT = w.reshape(CAP, E)
    hw = _ffn1(buf16, W1, b1, wT)
    y2 = y.astype(jnp.int32).reshape(1, 1)
    loss = _ffn2_loss(hw.reshape(1, E * DF), W2.reshape(E * DF, DM),
                      xmean, sw, b2, y2)
    return loss.reshape(())


# speedup vs baseline: 2.4371x; 2.4371x over previous
import functools

import jax
import jax.numpy as jnp
from jax import lax
from jax.experimental import pallas as pl
from jax.experimental.pallas import tpu as pltpu
from jax.experimental.pallas import tpu_sc as plsc

E = 8
TOPK = 2
DM = 1024
DF = 4096
CAP = 640
NSLOT = E * CAP
NC = 2
NS = 16
NW = NC * NS
SPW = NSLOT // NW
GCH = SPW // 2
L = 16

FT = 512
CT = 128
KT = 2048


NROW = 5248
TRASH = NSLOT
TPW = 2048 // NW


def _gate_body(x_ref, wg_ref, d1_ref, d2_ref, t1_ref, t2_ref, k1_ref, k2_ref,
               g1_ref, g2_ref, sw_ref, xmean_ref):
    xf = x_ref[...]
    T = xf.shape[0]
    logits = jnp.dot(xf, wg_ref[...], preferred_element_type=jnp.float32)
    m = jnp.max(logits, axis=-1, keepdims=True)
    ex = jnp.exp(logits - m)
    sm = ex / jnp.sum(ex, axis=-1, keepdims=True)
    iotaE = lax.broadcasted_iota(jnp.int32, (T, E), 1)
    g1v = jnp.max(sm, axis=-1, keepdims=True)
    e1 = jnp.min(jnp.where(sm >= g1v, iotaE, E), axis=-1, keepdims=True)
    sm2 = jnp.where(iotaE == e1, -1.0, sm)
    g2v = jnp.max(sm2, axis=-1, keepdims=True)
    e2 = jnp.min(jnp.where(sm2 >= g2v, iotaE, E), axis=-1, keepdims=True)
    ssum = g1v + g2v + 1e-9
    g1n = g1v / ssum
    g2n = g2v / ssum
    mask1 = (iotaE == e1).astype(jnp.float32)
    mask2 = (iotaE == e2).astype(jnp.float32)
    both = jnp.concatenate([mask1, mask2], axis=1)
    r = lax.broadcasted_iota(jnp.int32, (T, T), 0)
    c = lax.broadcasted_iota(jnp.int32, (T, T), 1)
    tri = (r >= c).astype(jnp.float32)
    cum = jnp.dot(tri, both, preferred_element_type=jnp.float32)
    cum1 = cum[:, :E]
    cum2 = cum[:, E:]
    n1 = cum1[T - 1:T, :]
    loc1 = cum1 - 1.0
    loc2 = cum2 - 1.0 + n1
    m1k = mask1 * (loc1 < CAP).astype(jnp.float32)
    m2k = mask2 * (loc2 < CAP).astype(jnp.float32)
    pos1 = jnp.sum(loc1 * m1k, axis=-1, keepdims=True).astype(jnp.int32)
    pos2 = jnp.sum(loc2 * m2k, axis=-1, keepdims=True).astype(jnp.int32)
    kept1 = jnp.sum(m1k, axis=-1, keepdims=True)
    kept2 = jnp.sum(m2k, axis=-1, keepdims=True)
    g1 = g1n * kept1
    g2 = g2n * kept2
    d1_ref[...] = e1 * CAP + pos1
    d2_ref[...] = e2 * CAP + pos2
    t1_ref[...] = pos1 * E + e1
    t2_ref[...] = pos2 * E + e2
    k1_ref[...] = kept1.astype(jnp.int32)
    k2_ref[...] = kept2.astype(jnp.int32)
    g1_ref[...] = g1
    g2_ref[...] = g2
    sw_ref[...] = jnp.sum(m1k * g1 + m2k * g2, axis=0, keepdims=True)
    xmean_ref[...] = jnp.sum(xf, axis=0, keepdims=True) * (1.0 / T)


def _gate(xf, Wg):
    T = xf.shape[0]
    f32, i32 = jnp.float32, jnp.int32
    outs = [((T, 1), i32), ((T, 1), i32), ((T, 1), i32), ((T, 1), i32),
            ((T, 1), i32), ((T, 1), i32), ((T, 1), f32), ((T, 1), f32),
            ((1, E), f32), ((1, DM), f32)]
    return pl.pallas_call(
        _gate_body,
        out_shape=[jax.ShapeDtypeStruct(s, d) for s, d in outs],
    )(xf, Wg)


def _dispatch_body(d1_hbm, d2_hbm, t1_hbm, t2_hbm, k1_hbm, k2_hbm,
                   g1_hbm, g2_hbm, x16_hbm,
                   buf_hbm, w_hbm,
                   idx1v, idx2v, k1v, k2v, rowsv,
                   dt1v, dt2v, kk1v, kk2v, gg1v, gg2v, wv,
                   sem1, sem2):
    cid = lax.axis_index("c")
    sid = lax.axis_index("s")
    wid = sid * NC + cid
    t0 = wid * TPW

    pltpu.sync_copy(x16_hbm.at[pl.ds(t0, TPW)], rowsv)
    pltpu.sync_copy(d1_hbm.at[pl.ds(t0, TPW)], idx1v)
    pltpu.sync_copy(d2_hbm.at[pl.ds(t0, TPW)], idx2v)
    pltpu.sync_copy(k1_hbm.at[pl.ds(t0, TPW)], k1v)
    pltpu.sync_copy(k2_hbm.at[pl.ds(t0, TPW)], k2v)
    for j in range(TPW // L):
        sl = pl.ds(j * L, L)
        idx1v[sl] = jnp.where(k1v[sl] != 0, idx1v[sl], TRASH)
        idx2v[sl] = jnp.where(k2v[sl] != 0, idx2v[sl], TRASH)
    cp1 = pltpu.async_copy(rowsv, buf_hbm.at[idx1v], sem1)
    cp2 = pltpu.async_copy(rowsv, buf_hbm.at[idx2v], sem2)

    @pl.when(jnp.logical_and(cid == 0, sid == 0))
    def _():
        T = dt1v.shape[0]
        pltpu.sync_copy(t1_hbm, dt1v)
        pltpu.sync_copy(t2_hbm, dt2v)
        pltpu.sync_copy(k1_hbm, kk1v)
        pltpu.sync_copy(k2_hbm, kk2v)
        pltpu.sync_copy(g1_hbm, gg1v)
        pltpu.sync_copy(g2_hbm, gg2v)
        zf = jnp.zeros((L,), jnp.float32)

        def zero_body(i, _):
            wv[pl.ds(i * L, L)] = zf
            return 0

        lax.fori_loop(0, NSLOT // L, zero_body, 0)

        def scat_body(i, _):
            sl = pl.ds(i * L, L)
            plsc.store_scatter(wv, [dt1v[sl]], gg1v[sl], mask=kk1v[sl] != 0)
            plsc.store_scatter(wv, [dt2v[sl]], gg2v[sl], mask=kk2v[sl] != 0)
            return 0

        lax.fori_loop(0, T // L, scat_body, 0)
        pltpu.sync_copy(wv, w_hbm)

    cp1.wait()
    cp2.wait()


def _dispatch(d1, d2, t1, t2, k1, k2, g1, g2, x16):
    T = x16.shape[0]
    mesh = plsc.VectorSubcoreMesh(core_axis_name="c", subcore_axis_name="s")
    f32, i32 = jnp.float32, jnp.int32
    kern = pl.kernel(
        _dispatch_body,
        out_type=[jax.ShapeDtypeStruct((NROW, DM), f32),
                  jax.ShapeDtypeStruct((NSLOT,), f32)],
        mesh=mesh,
        scratch_types=[
            pltpu.VMEM((TPW,), i32), pltpu.VMEM((TPW,), i32),
            pltpu.VMEM((TPW,), i32), pltpu.VMEM((TPW,), i32),
            pltpu.VMEM((TPW, DM), f32),
            pltpu.VMEM((T,), i32), pltpu.VMEM((T,), i32),
            pltpu.VMEM((T,), i32), pltpu.VMEM((T,), i32),
            pltpu.VMEM((T,), f32), pltpu.VMEM((T,), f32),
            pltpu.VMEM((NSLOT,), f32),
            pltpu.SemaphoreType.DMA, pltpu.SemaphoreType.DMA,
        ],
        compiler_params=pltpu.CompilerParams(needs_layout_passes=False),
    )
    return kern(d1, d2, t1, t2, k1, k2, g1, g2, x16)


def _ffn1_body(buf_ref, w1_ref, b1_ref, wT_ref, hw_ref, w1bf_ref):
    e = pl.program_id(0)
    ct = pl.program_id(1)

    @pl.when(ct == 0)
    def _():
        hw_ref[...] = jnp.zeros_like(hw_ref)
        w1bf_ref[...] = w1_ref[0].astype(jnp.bfloat16)

    x = buf_ref[...].astype(jnp.bfloat16)
    h = jnp.dot(x, w1bf_ref[...], preferred_element_type=jnp.float32)
    row = lax.broadcasted_iota(jnp.int32, (E, DF), 0)
    b1row = jnp.sum(jnp.where(row == e, b1_ref[...], 0.0), axis=0,
                    keepdims=True)
    h = jnp.maximum(h + b1row, 0.0)
    wall = wT_ref[pl.ds(ct * CT, CT), :]
    lane = lax.broadcasted_iota(jnp.int32, (CT, E), 1)
    wv = jnp.sum(jnp.where(lane == e, wall, 0.0), axis=1, keepdims=True)
    red = jnp.sum(jnp.where(wv > 0.0, h * wv, 0.0), axis=0, keepdims=True)
    hw_ref[...] += red.reshape(1, 1, DF)


def _ffn1(buf2d, W1, b1, wT):
    grid = (E, CAP // CT)
    return pl.pallas_call(
        _ffn1_body,
        grid=grid,
        in_specs=[
            pl.BlockSpec((CT, DM), lambda e, c: (e * (CAP // CT) + c, 0)),
            pl.BlockSpec((1, DM, DF), lambda e, c: (e, 0, 0)),
            pl.BlockSpec((E, DF), lambda e, c: (0, 0)),
            pl.BlockSpec((CAP, E), lambda e, c: (0, 0)),
        ],
        out_specs=pl.BlockSpec((1, 1, DF), lambda e, c: (e, 0, 0)),
        out_shape=jax.ShapeDtypeStruct((E, 1, DF), jnp.float32),
        scratch_shapes=[pltpu.VMEM((DM, DF), jnp.bfloat16)],
    )(buf2d, W1, b1, wT)


def _ffn2_body(hw_ref, w2_ref, xmean_ref, sw_ref, b2_ref, y_ref, out_ref,
               acc_ref):
    k = pl.program_id(0)
    nk = pl.num_programs(0)

    @pl.when(k == 0)
    def _():
        acc_ref[...] = jnp.zeros_like(acc_ref)

    acc_ref[...] += jnp.dot(hw_ref[...], w2_ref[...],
                            preferred_element_type=jnp.float32)

    @pl.when(k == nk - 1)
    def _():
        T = 2048.0
        bias = jnp.dot(sw_ref[...], b2_ref[...],
                       preferred_element_type=jnp.float32)
        sent = xmean_ref[...] + (acc_ref[...] + bias) * (1.0 / T)
        mm = jnp.max(sent)
        lse = jnp.log(jnp.sum(jnp.exp(sent - mm))) + mm
        col = lax.broadcasted_iota(jnp.int32, (1, DM), 1)
        tgt = jnp.sum(jnp.where(col == y_ref[0, 0], sent, 0.0))
        out_ref[0, 0] = lse - tgt


def _ffn2_loss(hwf, W2f, xmean, sw, b2, y2):
    nk = hwf.shape[1] // KT
    return pl.pallas_call(
        _ffn2_body,
        grid=(nk,),
        in_specs=[
            pl.BlockSpec((1, KT), lambda k: (0, k)),
            pl.BlockSpec((KT, DM), lambda k: (k, 0)),
            pl.BlockSpec((1, DM), lambda k: (0, 0)),
            pl.BlockSpec((1, E), lambda k: (0, 0)),
            pl.BlockSpec((E, DM), lambda k: (0, 0)),
            pl.BlockSpec(memory_space=pltpu.SMEM),
        ],
        out_specs=pl.BlockSpec(memory_space=pltpu.SMEM),
        out_shape=jax.ShapeDtypeStruct((1, 1), jnp.float32),
        scratch_shapes=[pltpu.VMEM((1, DM), jnp.float32)],
    )(hwf, W2f, xmean, sw, b2, y2)


def kernel(x, y, Wg, W1, b1, W2, b2):
    B, S, _ = x.shape
    T = B * S
    xf = x.reshape(T, DM)
    d1, d2, t1, t2, k1, k2, g1, g2, sw, xmean = _gate(xf, Wg)
    buf, w = _dispatch(d1.reshape(T), d2.reshape(T), t1.reshape(T),
                       t2.reshape(T), k1.reshape(T), k2.reshape(T),
                       g1.reshape(T), g2.reshape(T), xf)
    wT = w.reshape(CAP, E)
    hw = _ffn1(buf, W1, b1, wT)
    y2 = y.astype(jnp.int32).reshape(1, 1)
    loss = _ffn2_loss(hw.reshape(1, E * DF), W2.reshape(E * DF, DM),
                      xmean, sw, b2, y2)
    return loss.reshape(())


# speedup vs baseline: 2.7000x; 1.1079x over previous
import functools

import jax
import jax.numpy as jnp
from jax import lax
from jax.experimental import pallas as pl
from jax.experimental.pallas import tpu as pltpu
from jax.experimental.pallas import tpu_sc as plsc

E = 8
TOPK = 2
DM = 1024
DF = 4096
CAP = 640
NSLOT = E * CAP
NC = 2
NS = 16
NW = NC * NS
SPW = NSLOT // NW
GCH = SPW // 2
L = 16

FT = 512
CT = 128
KT = 2048


NROW = 5248
TRASH = NSLOT
TPW = 2048 // NW

import ml_dtypes as _mld
import numpy as _np
_TRI = _np.tril(_np.ones((2048, 2048), _np.float32)).astype(_mld.bfloat16)


def _gate_body(x_ref, wg_ref, tri_ref, d1_ref, d2_ref, t1_ref, t2_ref,
               k1_ref, k2_ref, g1_ref, g2_ref, sw_ref, xmean_ref):
    xf = x_ref[...]
    T = xf.shape[0]
    logits = jnp.dot(xf, wg_ref[...], preferred_element_type=jnp.float32)
    m = jnp.max(logits, axis=-1, keepdims=True)
    ex = jnp.exp(logits - m)
    sm = ex / jnp.sum(ex, axis=-1, keepdims=True)
    iotaE = lax.broadcasted_iota(jnp.int32, (T, E), 1)
    g1v = jnp.max(sm, axis=-1, keepdims=True)
    e1 = jnp.min(jnp.where(sm >= g1v, iotaE, E), axis=-1, keepdims=True)
    sm2 = jnp.where(iotaE == e1, -1.0, sm)
    g2v = jnp.max(sm2, axis=-1, keepdims=True)
    e2 = jnp.min(jnp.where(sm2 >= g2v, iotaE, E), axis=-1, keepdims=True)
    ssum = g1v + g2v + 1e-9
    g1n = g1v / ssum
    g2n = g2v / ssum
    mask1 = (iotaE == e1).astype(jnp.float32)
    mask2 = (iotaE == e2).astype(jnp.float32)
    both = jnp.concatenate([mask1, mask2], axis=1)
    cum = jnp.dot(tri_ref[...], both.astype(jnp.bfloat16),
                  preferred_element_type=jnp.float32)
    cum1 = cum[:, :E]
    cum2 = cum[:, E:]
    n1 = cum1[T - 1:T, :]
    loc1 = cum1 - 1.0
    loc2 = cum2 - 1.0 + n1
    m1k = mask1 * (loc1 < CAP).astype(jnp.float32)
    m2k = mask2 * (loc2 < CAP).astype(jnp.float32)
    pos1 = jnp.sum(loc1 * m1k, axis=-1, keepdims=True).astype(jnp.int32)
    pos2 = jnp.sum(loc2 * m2k, axis=-1, keepdims=True).astype(jnp.int32)
    kept1 = jnp.sum(m1k, axis=-1, keepdims=True)
    kept2 = jnp.sum(m2k, axis=-1, keepdims=True)
    g1 = g1n * kept1
    g2 = g2n * kept2
    d1_ref[...] = e1 * CAP + pos1
    d2_ref[...] = e2 * CAP + pos2
    t1_ref[...] = pos1 * E + e1
    t2_ref[...] = pos2 * E + e2
    k1_ref[...] = kept1.astype(jnp.int32)
    k2_ref[...] = kept2.astype(jnp.int32)
    g1_ref[...] = g1
    g2_ref[...] = g2
    sw_ref[...] = jnp.sum(m1k * g1 + m2k * g2, axis=0, keepdims=True)
    xmean_ref[...] = jnp.sum(xf, axis=0, keepdims=True) * (1.0 / T)


def _gate(xf, Wg, tri):
    T = xf.shape[0]
    f32, i32 = jnp.float32, jnp.int32
    outs = [((T, 1), i32), ((T, 1), i32), ((T, 1), i32), ((T, 1), i32),
            ((T, 1), i32), ((T, 1), i32), ((T, 1), f32), ((T, 1), f32),
            ((1, E), f32), ((1, DM), f32)]
    return pl.pallas_call(
        _gate_body,
        out_shape=[jax.ShapeDtypeStruct(s, d) for s, d in outs],
    )(xf, Wg, tri)


def _dispatch_body(d1_hbm, d2_hbm, t1_hbm, t2_hbm, k1_hbm, k2_hbm,
                   g1_hbm, g2_hbm, x16_hbm,
                   buf_hbm, w_hbm,
                   idx1v, idx2v, k1v, k2v, rowsv,
                   dt1v, dt2v, kk1v, kk2v, gg1v, gg2v, wv,
                   sem1, sem2):
    cid = lax.axis_index("c")
    sid = lax.axis_index("s")
    wid = sid * NC + cid
    t0 = wid * TPW

    pltpu.sync_copy(x16_hbm.at[pl.ds(t0, TPW)], rowsv)
    pltpu.sync_copy(d1_hbm.at[pl.ds(t0, TPW)], idx1v)
    pltpu.sync_copy(d2_hbm.at[pl.ds(t0, TPW)], idx2v)
    pltpu.sync_copy(k1_hbm.at[pl.ds(t0, TPW)], k1v)
    pltpu.sync_copy(k2_hbm.at[pl.ds(t0, TPW)], k2v)
    for j in range(TPW // L):
        sl = pl.ds(j * L, L)
        idx1v[sl] = jnp.where(k1v[sl] != 0, idx1v[sl], TRASH)
        idx2v[sl] = jnp.where(k2v[sl] != 0, idx2v[sl], TRASH)
    cp1 = pltpu.async_copy(rowsv, buf_hbm.at[idx1v], sem1)
    cp2 = pltpu.async_copy(rowsv, buf_hbm.at[idx2v], sem2)

    @pl.when(jnp.logical_and(cid == 0, sid == 0))
    def _():
        T = dt1v.shape[0]
        pltpu.sync_copy(t1_hbm, dt1v)
        pltpu.sync_copy(t2_hbm, dt2v)
        pltpu.sync_copy(k1_hbm, kk1v)
        pltpu.sync_copy(k2_hbm, kk2v)
        pltpu.sync_copy(g1_hbm, gg1v)
        pltpu.sync_copy(g2_hbm, gg2v)
        zf = jnp.zeros((L,), jnp.float32)

        def zero_body(i, _):
            wv[pl.ds(i * L, L)] = zf
            return 0

        lax.fori_loop(0, NSLOT // L, zero_body, 0)

        def scat_body(i, _):
            sl = pl.ds(i * L, L)
            plsc.store_scatter(wv, [dt1v[sl]], gg1v[sl], mask=kk1v[sl] != 0)
            plsc.store_scatter(wv, [dt2v[sl]], gg2v[sl], mask=kk2v[sl] != 0)
            return 0

        lax.fori_loop(0, T // L, scat_body, 0)
        pltpu.sync_copy(wv, w_hbm)

    cp1.wait()
    cp2.wait()


def _dispatch(d1, d2, t1, t2, k1, k2, g1, g2, x16):
    T = x16.shape[0]
    mesh = plsc.VectorSubcoreMesh(core_axis_name="c", subcore_axis_name="s")
    f32, i32 = jnp.float32, jnp.int32
    kern = pl.kernel(
        _dispatch_body,
        out_type=[jax.ShapeDtypeStruct((NROW, DM), f32),
                  jax.ShapeDtypeStruct((NSLOT,), f32)],
        mesh=mesh,
        scratch_types=[
            pltpu.VMEM((TPW,), i32), pltpu.VMEM((TPW,), i32),
            pltpu.VMEM((TPW,), i32), pltpu.VMEM((TPW,), i32),
            pltpu.VMEM((TPW, DM), f32),
            pltpu.VMEM((T,), i32), pltpu.VMEM((T,), i32),
            pltpu.VMEM((T,), i32), pltpu.VMEM((T,), i32),
            pltpu.VMEM((T,), f32), pltpu.VMEM((T,), f32),
            pltpu.VMEM((NSLOT,), f32),
            pltpu.SemaphoreType.DMA, pltpu.SemaphoreType.DMA,
        ],
        compiler_params=pltpu.CompilerParams(needs_layout_passes=False),
    )
    return kern(d1, d2, t1, t2, k1, k2, g1, g2, x16)


def _ffn_body(buf_ref, w1_ref, b1_ref, wT_ref, w2_ref, xmean_ref, sw_ref,
              b2_ref, y_ref, out_ref, xbf_ref, sent_ref):
    e = pl.program_id(0)
    c = pl.program_id(1)

    @pl.when(jnp.logical_and(e == 0, c == 0))
    def _():
        sent_ref[...] = jnp.zeros_like(sent_ref)

    @pl.when(c == 0)
    def _():
        xbf_ref[...] = buf_ref[...].astype(jnp.bfloat16)

    w1c = w1_ref[0].astype(jnp.bfloat16)
    h = jnp.dot(xbf_ref[...], w1c, preferred_element_type=jnp.float32)
    b1all = b1_ref[:, pl.ds(c * FT, FT)]
    row = lax.broadcasted_iota(jnp.int32, (E, FT), 0)
    b1row = jnp.sum(jnp.where(row == e, b1all, 0.0), axis=0, keepdims=True)
    h = jnp.maximum(h + b1row, 0.0)
    wall = wT_ref[...]
    lane = lax.broadcasted_iota(jnp.int32, (CAP, E), 1)
    wv = jnp.sum(jnp.where(lane == e, wall, 0.0), axis=1, keepdims=True)
    red = jnp.sum(jnp.where(wv > 0.0, h * wv, 0.0), axis=0, keepdims=True)
    sent_ref[...] += jnp.dot(red, w2_ref[0],
                             preferred_element_type=jnp.float32)

    @pl.when(jnp.logical_and(e == E - 1, c == DF // FT - 1))
    def _():
        T = 2048.0
        bias = jnp.dot(sw_ref[...], b2_ref[...],
                       preferred_element_type=jnp.float32)
        sent = xmean_ref[...] + (sent_ref[...] + bias) * (1.0 / T)
        mm = jnp.max(sent)
        lse = jnp.log(jnp.sum(jnp.exp(sent - mm))) + mm
        col = lax.broadcasted_iota(jnp.int32, (1, DM), 1)
        tgt = jnp.sum(jnp.where(col == y_ref[0, 0], sent, 0.0))
        out_ref[0, 0] = lse - tgt


def _ffn_loss(buf2d, W1, b1, wT, W2, xmean, sw, b2, y2):
    grid = (E, DF // FT)
    return pl.pallas_call(
        _ffn_body,
        grid=grid,
        in_specs=[
            pl.BlockSpec((CAP, DM), lambda e, c: (e, 0)),
            pl.BlockSpec((1, DM, FT), lambda e, c: (e, 0, c)),
            pl.BlockSpec((E, DF), lambda e, c: (0, 0)),
            pl.BlockSpec((CAP, E), lambda e, c: (0, 0)),
            pl.BlockSpec((1, FT, DM), lambda e, c: (e, c, 0)),
            pl.BlockSpec((1, DM), lambda e, c: (0, 0)),
            pl.BlockSpec((1, E), lambda e, c: (0, 0)),
            pl.BlockSpec((E, DM), lambda e, c: (0, 0)),
            pl.BlockSpec(memory_space=pltpu.SMEM),
        ],
        out_specs=pl.BlockSpec(memory_space=pltpu.SMEM),
        out_shape=jax.ShapeDtypeStruct((1, 1), jnp.float32),
        scratch_shapes=[pltpu.VMEM((CAP, DM), jnp.bfloat16),
                        pltpu.VMEM((1, DM), jnp.float32)],
    )(buf2d, W1, b1, wT, W2, xmean, sw, b2, y2)


def kernel(x, y, Wg, W1, b1, W2, b2):
    B, S, _ = x.shape
    T = B * S
    xf = x.reshape(T, DM)
    tri = jnp.asarray(_TRI)
    d1, d2, t1, t2, k1, k2, g1, g2, sw, xmean = _gate(xf, Wg, tri)
    buf, w = _dispatch(d1.reshape(T), d2.reshape(T), t1.reshape(T),
                       t2.reshape(T), k1.reshape(T), k2.reshape(T),
                       g1.reshape(T), g2.reshape(T), xf)
    wT = w.reshape(CAP, E)
    y2 = y.astype(jnp.int32).reshape(1, 1)
    loss = _ffn_loss(buf, W1, b1, wT, W2, xmean, sw, b2, y2)
    return loss.reshape(())


# speedup vs baseline: 2.8292x; 1.0479x over previous
import functools

import jax
import jax.numpy as jnp
import ml_dtypes as _mld
import numpy as _np
from jax import lax
from jax.experimental import pallas as pl
from jax.experimental.pallas import tpu as pltpu
from jax.experimental.pallas import tpu_sc as plsc

E = 8
DM = 1024
DF = 4096
CAP = 640
NSLOT = E * CAP
NC = 2
NS = 16
NW = NC * NS
L = 16
TSEQ = 2048
TPW = TSEQ // NW
NROW = 5248
TRASH = NSLOT
FT = 512

_TRI = _np.tril(_np.ones((TSEQ, TSEQ), _np.float32)).astype(_mld.bfloat16)


def _gate_body(x_ref, wg_ref, tri_ref, d1_ref, d2_ref, g1_ref, g2_ref,
               fill_ref, sw_ref, xmean_ref):
    xf = x_ref[...]
    T = xf.shape[0]
    logits = jnp.dot(xf, wg_ref[...], preferred_element_type=jnp.float32)
    m = jnp.max(logits, axis=-1, keepdims=True)
    ex = jnp.exp(logits - m)
    sm = ex / jnp.sum(ex, axis=-1, keepdims=True)
    iotaE = lax.broadcasted_iota(jnp.int32, (T, E), 1)
    g1v = jnp.max(sm, axis=-1, keepdims=True)
    e1 = jnp.min(jnp.where(sm >= g1v, iotaE, E), axis=-1, keepdims=True)
    sm2 = jnp.where(iotaE == e1, -1.0, sm)
    g2v = jnp.max(sm2, axis=-1, keepdims=True)
    e2 = jnp.min(jnp.where(sm2 >= g2v, iotaE, E), axis=-1, keepdims=True)
    ssum = g1v + g2v + 1e-9
    mask1 = (iotaE == e1).astype(jnp.float32)
    mask2 = (iotaE == e2).astype(jnp.float32)
    both = jnp.concatenate([mask1, mask2], axis=1)
    cum = jnp.dot(tri_ref[...], both.astype(jnp.bfloat16),
                  preferred_element_type=jnp.float32)
    cum1 = cum[:, :E]
    cum2 = cum[:, E:]
    n1 = cum1[T - 1:T, :]
    n2 = cum2[T - 1:T, :]
    loc1 = cum1 - 1.0
    loc2 = cum2 - 1.0 + n1
    m1k = mask1 * (loc1 < CAP).astype(jnp.float32)
    m2k = mask2 * (loc2 < CAP).astype(jnp.float32)
    pos1 = jnp.sum(loc1 * m1k, axis=-1, keepdims=True).astype(jnp.int32)
    pos2 = jnp.sum(loc2 * m2k, axis=-1, keepdims=True).astype(jnp.int32)
    kept1 = jnp.sum(m1k, axis=-1, keepdims=True)
    kept2 = jnp.sum(m2k, axis=-1, keepdims=True)
    g1 = (g1v / ssum) * kept1
    g2 = (g2v / ssum) * kept2
    d1 = e1 * CAP + pos1
    d2 = e2 * CAP + pos2
    d1_ref[...] = jnp.where(kept1 > 0.0, d1, TRASH)
    d2_ref[...] = jnp.where(kept2 > 0.0, d2, TRASH)
    g1_ref[...] = g1
    g2_ref[...] = g2
    fill_ref[...] = jnp.minimum(n1 + n2, float(CAP))
    sw_ref[...] = jnp.sum(m1k * g1 + m2k * g2, axis=0, keepdims=True)
    xmean_ref[...] = jnp.sum(xf, axis=0, keepdims=True) * (1.0 / T)


def _gate(xf, Wg, tri):
    T = xf.shape[0]
    f32, i32 = jnp.float32, jnp.int32
    outs = [((T, 1), i32), ((T, 1), i32), ((T, 1), f32), ((T, 1), f32),
            ((1, E), f32), ((1, E), f32), ((1, DM), f32)]
    return pl.pallas_call(
        _gate_body,
        out_shape=[jax.ShapeDtypeStruct(s, d) for s, d in outs],
    )(xf, Wg, tri)


def _dispatch_body(d1_hbm, d2_hbm, g1_hbm, g2_hbm, xf_hbm,
                   buf_hbm, w_hbm,
                   idx1v, idx2v, rowsv, dd1v, dd2v, gg1v, gg2v, wv,
                   sem1, sem2):
    cid = lax.axis_index("c")
    sid = lax.axis_index("s")
    wid = sid * NC + cid
    t0 = wid * TPW

    pltpu.sync_copy(xf_hbm.at[pl.ds(t0, TPW)], rowsv)
    pltpu.sync_copy(d1_hbm.at[pl.ds(t0, TPW)], idx1v)
    pltpu.sync_copy(d2_hbm.at[pl.ds(t0, TPW)], idx2v)
    cp1 = pltpu.async_copy(rowsv, buf_hbm.at[idx1v], sem1)
    cp2 = pltpu.async_copy(rowsv, buf_hbm.at[idx2v], sem2)

    @pl.when(jnp.logical_and(cid == 0, sid == 0))
    def _():
        T = dd1v.shape[0]
        pltpu.sync_copy(d1_hbm, dd1v)
        pltpu.sync_copy(d2_hbm, dd2v)
        pltpu.sync_copy(g1_hbm, gg1v)
        pltpu.sync_copy(g2_hbm, gg2v)
        zf = jnp.zeros((L,), jnp.float32)

        def zero_body(i, _):
            wv[pl.ds(i * L, L)] = zf
            return 0

        lax.fori_loop(0, NSLOT // L, zero_body, 0, unroll=4)

        def scat_body(i, _):
            sl = pl.ds(i * L, L)
            i1 = dd1v[sl]
            i2 = dd2v[sl]
            plsc.store_scatter(wv, [i1], gg1v[sl], mask=i1 < NSLOT)
            plsc.store_scatter(wv, [i2], gg2v[sl], mask=i2 < NSLOT)
            return 0

        lax.fori_loop(0, T // L, scat_body, 0, unroll=4)
        pltpu.sync_copy(wv, w_hbm)

    cp1.wait()
    cp2.wait()


def _dispatch(d1, d2, g1, g2, xf):
    T = xf.shape[0]
    mesh = plsc.VectorSubcoreMesh(core_axis_name="c", subcore_axis_name="s")
    f32, i32 = jnp.float32, jnp.int32
    kern = pl.kernel(
        _dispatch_body,
        out_type=[jax.ShapeDtypeStruct((NROW, DM), f32),
                  jax.ShapeDtypeStruct((NSLOT,), f32)],
        mesh=mesh,
        scratch_types=[
            pltpu.VMEM((TPW,), i32), pltpu.VMEM((TPW,), i32),
            pltpu.VMEM((TPW, DM), f32),
            pltpu.VMEM((T,), i32), pltpu.VMEM((T,), i32),
            pltpu.VMEM((T,), f32), pltpu.VMEM((T,), f32),
            pltpu.VMEM((NSLOT,), f32),
            pltpu.SemaphoreType.DMA, pltpu.SemaphoreType.DMA,
        ],
        compiler_params=pltpu.CompilerParams(needs_layout_passes=False),
    )
    return kern(d1, d2, g1, g2, xf)


def _ffn_body(buf_ref, w1_ref, b1_ref, w2d_ref, fill_ref, w2_ref, xmean_ref,
              sw_ref, b2_ref, y_ref, out_ref, xbf_ref, sent_ref):
    e = pl.program_id(0)
    c = pl.program_id(1)

    @pl.when(jnp.logical_and(e == 0, c == 0))
    def _():
        sent_ref[...] = jnp.zeros_like(sent_ref)

    rowE = lax.broadcasted_iota(jnp.int32, (1, E), 1)

    @pl.when(c == 0)
    def _():
        fe = jnp.sum(jnp.where(rowE == e, fill_ref[...], 0.0))
        rows = lax.broadcasted_iota(jnp.int32, (CAP, 1), 0)
        xbf_ref[...] = jnp.where(rows.astype(jnp.float32) < fe,
                                 buf_ref[...], 0.0).astype(jnp.bfloat16)

    w1c = w1_ref[0].astype(jnp.bfloat16)
    h = jnp.dot(xbf_ref[...], w1c, preferred_element_type=jnp.float32)
    b1all = b1_ref[:, pl.ds(c * FT, FT)]
    rowEF = lax.broadcasted_iota(jnp.int32, (E, FT), 0)
    b1row = jnp.sum(jnp.where(rowEF == e, b1all, 0.0), axis=0, keepdims=True)
    h = jnp.maximum(h + b1row, 0.0)
    rowEC = lax.broadcasted_iota(jnp.int32, (E, CAP), 0)
    we = jnp.sum(jnp.where(rowEC == e, w2d_ref[...], 0.0), axis=0,
                 keepdims=True)
    red = jnp.dot(we, h, preferred_element_type=jnp.float32)
    sent_ref[...] += jnp.dot(red, w2_ref[0],
                             preferred_element_type=jnp.float32)

    @pl.when(jnp.logical_and(e == E - 1, c == DF // FT - 1))
    def _():
        bias = jnp.dot(sw_ref[...], b2_ref[...],
                       preferred_element_type=jnp.float32)
        sent = xmean_ref[...] + (sent_ref[...] + bias) * (1.0 / TSEQ)
        mm = jnp.max(sent)
        lse = jnp.log(jnp.sum(jnp.exp(sent - mm))) + mm
        col = lax.broadcasted_iota(jnp.int32, (1, DM), 1)
        tgt = jnp.sum(jnp.where(col == y_ref[0, 0], sent, 0.0))
        out_ref[0, 0] = lse - tgt


def _ffn_loss(buf2d, W1, b1, w2d, fill, W2, xmean, sw, b2, y2):
    grid = (E, DF // FT)
    return pl.pallas_call(
        _ffn_body,
        grid=grid,
        in_specs=[
            pl.BlockSpec((CAP, DM), lambda e, c: (e, 0)),
            pl.BlockSpec((1, DM, FT), lambda e, c: (e, 0, c)),
            pl.BlockSpec((E, DF), lambda e, c: (0, 0)),
            pl.BlockSpec((E, CAP), lambda e, c: (0, 0)),
            pl.BlockSpec((1, E), lambda e, c: (0, 0)),
            pl.BlockSpec((1, FT, DM), lambda e, c: (e, c, 0)),
            pl.BlockSpec((1, DM), lambda e, c: (0, 0)),
            pl.BlockSpec((1, E), lambda e, c: (0, 0)),
            pl.BlockSpec((E, DM), lambda e, c: (0, 0)),
            pl.BlockSpec(memory_space=pltpu.SMEM),
        ],
        out_specs=pl.BlockSpec(memory_space=pltpu.SMEM),
        out_shape=jax.ShapeDtypeStruct((1, 1), jnp.float32),
        scratch_shapes=[pltpu.VMEM((CAP, DM), jnp.bfloat16),
                        pltpu.VMEM((1, DM), jnp.float32)],
    )(buf2d, W1, b1, w2d, fill, W2, xmean, sw, b2, y2)


def kernel(x, y, Wg, W1, b1, W2, b2):
    B, S, _ = x.shape
    T = B * S
    xf = x.reshape(T, DM)
    tri = jnp.asarray(_TRI)
    d1, d2, g1, g2, fill, sw, xmean = _gate(xf, Wg, tri)
    buf, w = _dispatch(d1.reshape(T), d2.reshape(T),
                       g1.reshape(T), g2.reshape(T), xf)
    w2d = w.reshape(E, CAP)
    y2 = y.astype(jnp.int32).reshape(1, 1)
    loss = _ffn_loss(buf, W1, b1, w2d, fill, W2, xmean, sw, b2, y2)
    return loss.reshape(())
